# baseline (speedup 1.0000x reference)
import math

import jax
import jax.numpy as jnp
from jax.experimental import pallas as pl
from jax.experimental.pallas import tpu as pltpu

_D = 768
_H = 12
_HKV = 4
_DH = 64
_HI = 4
_DI = 32
_KSEL = 64
_ROPE_BASE = 10000.0
_S = 2048
_BLK = 256
_NREP = _H // _HKV
_LN_BASE = math.log(_ROPE_BASE)


def _mm(a, b):
    return jax.lax.dot_general(
        a, b, (((1,), (0,)), ((), ())),
        preferred_element_type=jnp.float32)


def _bmm(a, b):
    return jax.lax.dot_general(
        a.astype(jnp.bfloat16), b.astype(jnp.bfloat16),
        (((1,), (0,)), ((), ())), preferred_element_type=jnp.float32)


def _bmm_t(a, b):
    return jax.lax.dot_general(
        a.astype(jnp.bfloat16), b.astype(jnp.bfloat16),
        (((1,), (1,)), ((), ())), preferred_element_type=jnp.float32)


def _mm_t(a, b):
    return jax.lax.dot_general(
        a, b, (((1,), (1,)), ((), ())),
        preferred_element_type=jnp.float32)


def _proj_kernel(hs_ref, wq_ref, wk_ref, wv_ref, wqi_ref, wki_ref,
                 wgi_ref, wgv_ref, bgv_ref, wgo_ref, bgo_ref,
                 q_ref, k_ref, v_ref, qi_ref, ki_ref, gi_ref, go_ref):
    i = pl.program_id(0)
    hs = hs_ref[...]

    j = jax.lax.broadcasted_iota(jnp.int32, (_BLK, _DI), 1).astype(jnp.float32)
    pos = (jax.lax.broadcasted_iota(jnp.int32, (_BLK, _DI), 0)
           + i * _BLK).astype(jnp.float32)
    f = pos * jnp.exp(j * jnp.float32(-_LN_BASE / _DI))
    cos_f = jnp.cos(f)
    sin_f = jnp.sin(f)

    def rope(x, nheads):
        parts = []
        for h in range(nheads):
            x1 = x[:, h * _DH:h * _DH + _DI]
            x2 = x[:, h * _DH + _DI:(h + 1) * _DH]
            parts.append(x1 * cos_f - x2 * sin_f)
            parts.append(x2 * cos_f + x1 * sin_f)
        return jnp.concatenate(parts, axis=1)

    q_ref[...] = rope(_mm(hs, wq_ref[...]), _H)
    k_ref[...] = rope(_mm(hs, wk_ref[...]), _HKV)

    v = _mm(hs, wv_ref[...])
    gv = jax.nn.sigmoid(_mm(hs, wgv_ref[...]) + bgv_ref[...])
    v_ref[...] = jnp.concatenate(
        [v[:, h * _DH:(h + 1) * _DH] * gv[:, h:h + 1] for h in range(_HKV)],
        axis=1)

    qi_ref[...] = _mm(hs, wqi_ref[...])
    ki_ref[...] = _mm(hs, wki_ref[...])
    gi_ref[...] = jax.nn.sigmoid(_mm(hs, wgi_ref[...]))
    go_ref[...] = jax.nn.sigmoid(_mm(hs, wgo_ref[...]) + bgo_ref[...])


def _attn_kernel(q_ref, qi_ref, gi_ref, go_ref, ki_ref, k_ref, v_ref,
                 wo_ref, out_ref):
    i = pl.program_id(0)
    qpos = jax.lax.broadcasted_iota(jnp.int32, (_BLK, _S), 0) + i * _BLK
    col = jax.lax.broadcasted_iota(jnp.int32, (_BLK, _S), 1)
    causal = col <= qpos

    qi = qi_ref[...]
    ki = ki_ref[...]
    gi = gi_ref[...]
    inv_sqrt_di = jnp.float32(1.0 / math.sqrt(_DI))
    scores = jnp.zeros((_BLK, _S), jnp.float32)
    for h in range(_HI):
        raw = _mm_t(qi[:, h * _DI:(h + 1) * _DI],
                    ki[:, h * _DI:(h + 1) * _DI]) * inv_sqrt_di
        act = jax.nn.sigmoid(raw).astype(jnp.bfloat16).astype(jnp.float32)
        gih = gi[:, h:h + 1].astype(jnp.bfloat16).astype(jnp.float32)
        scores = scores + act * gih
    scores = jnp.where(causal, scores, 0.0)
    bits = jax.lax.bitcast_convert_type(scores, jnp.int32)

    p16 = (bits >> 16).astype(jnp.int16)

    def body1(_, carry):
        lo, hi = carry
        mid = (lo & hi) + ((lo ^ hi) >> 1)
        cnt = jnp.sum((p16 >= mid.astype(jnp.int16)).astype(jnp.int16),
                      axis=1, keepdims=True).astype(jnp.int32)
        ge = cnt >= _KSEL
        return jnp.where(ge, mid, lo), jnp.where(ge, hi, mid)

    t16, _ = jax.lax.fori_loop(
        0, 15, body1,
        (jnp.zeros((_BLK, 1), jnp.int32),
         jnp.full((_BLK, 1), jnp.int32(0x4100))))

    g = jnp.sum((p16 > t16.astype(jnp.int16)).astype(jnp.int16),
                axis=1, keepdims=True).astype(jnp.int32)
    w15 = jnp.where(p16 == t16.astype(jnp.int16),
                    ((jnp.right_shift(bits, 1) & 0x7FFF) - 1)
                    .astype(jnp.int16),
                    jnp.int16(-2))

    def body2(_, carry):
        lo, hi = carry
        mid = (lo & hi) + ((lo ^ hi) >> 1)
        cnt = jnp.sum((w15 >= mid.astype(jnp.int16)).astype(jnp.int16),
                      axis=1, keepdims=True).astype(jnp.int32)
        ge = (g + cnt) >= _KSEL
        return jnp.where(ge, mid, lo), jnp.where(ge, hi, mid)

    t2s, _ = jax.lax.fori_loop(
        0, 15, body2,
        (jnp.full((_BLK, 1), jnp.int32(-1)),
         jnp.full((_BLK, 1), jnp.int32(0x7FFF))))

    x0 = (t16 << 16) | ((t2s + 1) << 1)
    c1 = jnp.sum((bits >= (x0 + 1)).astype(jnp.int32), axis=1, keepdims=True)
    lo = jnp.where(c1 >= _KSEL, x0 + 1, x0)
    gt = bits > lo
    eq = ((bits == lo) & causal).astype(jnp.int32)
    csum = eq
    for sh in range(11):
        rolled = jnp.roll(csum, 1 << sh, axis=1)
        csum = csum + jnp.where(col >= (1 << sh), rolled, 0)
    need = _KSEL - jnp.sum((gt & causal).astype(jnp.int32), axis=1,
                           keepdims=True)
    selmask = (gt | ((eq > 0) & (csum <= need))) & causal

    q = q_ref[...]
    k = k_ref[...]
    v = v_ref[...]
    go = go_ref[...]
    scale = jnp.float32(1.0 / math.sqrt(_DH))
    neg = jnp.float32(-jnp.inf)
    outs = []
    for h in range(_H):
        hkv = h // _NREP
        att = _mm_t(q[:, h * _DH:(h + 1) * _DH],
                    k[:, hkv * _DH:(hkv + 1) * _DH]) * scale
        att = jnp.where(selmask, att, neg)
        m = jnp.max(att, axis=1, keepdims=True)
        p = jnp.exp(att - m)
        w = p / jnp.sum(p, axis=1, keepdims=True)
        oh = _mm(w, v[:, hkv * _DH:(hkv + 1) * _DH])
        outs.append(oh * go[:, h:h + 1])
    o = jnp.concatenate(outs, axis=1)
    out_ref[...] = _mm(o, wo_ref[...])


def _full(shape):
    return pl.BlockSpec(shape, lambda i: (0,) * len(shape))


def _rows(width):
    return pl.BlockSpec((_BLK, width), lambda i: (i, 0))


def kernel(hidden_states, positions, Wq, Wk, Wv, Wo, Wqi, Wki, Wgi, Wgv,
           bgv, Wgo, bgo, interpret=False):
    del positions
    b, s, d = hidden_states.shape
    hs = hidden_states.reshape(s, d)
    grid = (s // _BLK,)

    q, k, v, qi, ki, gi, go = pl.pallas_call(
        _proj_kernel,
        grid=grid,
        in_specs=[
            _rows(_D), _full((_D, _H * _DH)), _full((_D, _HKV * _DH)),
            _full((_D, _HKV * _DH)), _full((_D, _HI * _DI)),
            _full((_D, _HI * _DI)), _full((_D, _HI)), _full((_D, _HKV)),
            _full((1, _HKV)), _full((_D, _H)), _full((1, _H)),
        ],
        out_specs=[
            _rows(_H * _DH), _rows(_HKV * _DH), _rows(_HKV * _DH),
            _rows(_HI * _DI), _rows(_HI * _DI), _rows(_HI), _rows(_H),
        ],
        out_shape=[
            jax.ShapeDtypeStruct((s, _H * _DH), jnp.float32),
            jax.ShapeDtypeStruct((s, _HKV * _DH), jnp.float32),
            jax.ShapeDtypeStruct((s, _HKV * _DH), jnp.float32),
            jax.ShapeDtypeStruct((s, _HI * _DI), jnp.float32),
            jax.ShapeDtypeStruct((s, _HI * _DI), jnp.float32),
            jax.ShapeDtypeStruct((s, _HI), jnp.float32),
            jax.ShapeDtypeStruct((s, _H), jnp.float32),
        ],
        compiler_params=pltpu.CompilerParams(
            dimension_semantics=("parallel",)),
        interpret=interpret,
    )(hs, Wq, Wk, Wv, Wqi, Wki, Wgi, Wgv, bgv.reshape(1, _HKV), Wgo,
      bgo.reshape(1, _H))

    out = pl.pallas_call(
        _attn_kernel,
        grid=grid,
        in_specs=[
            _rows(_H * _DH), _rows(_HI * _DI), _rows(_HI), _rows(_H),
            _full((s, _HI * _DI)), _full((s, _HKV * _DH)),
            _full((s, _HKV * _DH)), _full((_H * _DH, _D)),
        ],
        out_specs=_rows(_D),
        out_shape=jax.ShapeDtypeStruct((s, _D), jnp.float32),
        compiler_params=pltpu.CompilerParams(
            dimension_semantics=("parallel",)),
        interpret=interpret,
    )(q, qi, gi, go, ki, k, v, Wo)

    return out.reshape(b, s, d)


# speedup vs baseline: 1.3884x; 1.3884x over previous
import math

import jax
import jax.numpy as jnp
from jax.experimental import pallas as pl
from jax.experimental.pallas import tpu as pltpu

_D = 768
_H = 12
_HKV = 4
_DH = 64
_HI = 4
_DI = 32
_KSEL = 64
_ROPE_BASE = 10000.0
_S = 2048
_BLK = 256
_NREP = _H // _HKV
_LN_BASE = math.log(_ROPE_BASE)


def _mm(a, b):
    return jax.lax.dot_general(
        a, b, (((1,), (0,)), ((), ())),
        preferred_element_type=jnp.float32)


def _bmm(a, b):
    return jax.lax.dot_general(
        a.astype(jnp.bfloat16), b.astype(jnp.bfloat16),
        (((1,), (0,)), ((), ())), preferred_element_type=jnp.float32)


def _bmm_t(a, b):
    return jax.lax.dot_general(
        a.astype(jnp.bfloat16), b.astype(jnp.bfloat16),
        (((1,), (1,)), ((), ())), preferred_element_type=jnp.float32)


def _mm_t(a, b):
    return jax.lax.dot_general(
        a, b, (((1,), (1,)), ((), ())),
        preferred_element_type=jnp.float32)


def _proj_kernel(hs_ref, wq_ref, wk_ref, wv_ref, wqi_ref, wki_ref,
                 wgi_ref, wgv_ref, bgv_ref, wgo_ref, bgo_ref,
                 q_ref, k_ref, v_ref, qi_ref, ki_ref, gi_ref, go_ref):
    i = pl.program_id(0)
    hs = hs_ref[...]

    j = jax.lax.broadcasted_iota(jnp.int32, (_BLK, _DI), 1).astype(jnp.float32)
    pos = (jax.lax.broadcasted_iota(jnp.int32, (_BLK, _DI), 0)
           + i * _BLK).astype(jnp.float32)
    f = pos * jnp.exp(j * jnp.float32(-_LN_BASE / _DI))
    cos_f = jnp.cos(f)
    sin_f = jnp.sin(f)

    def rope(x, nheads):
        parts = []
        for h in range(nheads):
            x1 = x[:, h * _DH:h * _DH + _DI]
            x2 = x[:, h * _DH + _DI:(h + 1) * _DH]
            parts.append(x1 * cos_f - x2 * sin_f)
            parts.append(x2 * cos_f + x1 * sin_f)
        return jnp.concatenate(parts, axis=1)

    q_ref[...] = rope(_mm(hs, wq_ref[...]), _H)
    k_ref[...] = rope(_mm(hs, wk_ref[...]), _HKV)

    v = _mm(hs, wv_ref[...])
    gv = jax.nn.sigmoid(_mm(hs, wgv_ref[...]) + bgv_ref[...])
    v_ref[...] = jnp.concatenate(
        [v[:, h * _DH:(h + 1) * _DH] * gv[:, h:h + 1] for h in range(_HKV)],
        axis=1)

    qi_ref[...] = _mm(hs, wqi_ref[...])
    ki_ref[...] = _mm(hs, wki_ref[...])
    gi_ref[...] = jax.nn.sigmoid(_mm(hs, wgi_ref[...]))
    go_ref[...] = jax.nn.sigmoid(_mm(hs, wgo_ref[...]) + bgo_ref[...])


def _attn_kernel(q_ref, qi_ref, gi_ref, go_ref, ki_ref, k_ref, v_ref,
                 wo_ref, out_ref):
    i = pl.program_id(0)
    kpos = jax.lax.broadcasted_iota(jnp.int32, (_S, _BLK), 0)
    qpos = jax.lax.broadcasted_iota(jnp.int32, (_S, _BLK), 1) + i * _BLK
    causal = kpos <= qpos

    qi = qi_ref[...]
    ki = ki_ref[...]
    gi_t = jnp.transpose(gi_ref[...])
    inv_sqrt_di = jnp.float32(1.0 / math.sqrt(_DI))
    scores = jnp.zeros((_S, _BLK), jnp.float32)
    for h in range(_HI):
        raw = _mm_t(ki[:, h * _DI:(h + 1) * _DI],
                    qi[:, h * _DI:(h + 1) * _DI]) * inv_sqrt_di
        act = jax.nn.sigmoid(raw).astype(jnp.bfloat16).astype(jnp.float32)
        gih = gi_t[h:h + 1, :].astype(jnp.bfloat16).astype(jnp.float32)
        scores = scores + act * gih
    scores = jnp.where(causal, scores, 0.0)
    bits = jax.lax.bitcast_convert_type(scores, jnp.int32)

    lo0 = jnp.zeros((1, _BLK), jnp.int32)
    hi0 = jnp.full((1, _BLK), jnp.int32(0x41000000))

    def body(_, carry):
        lo, hi = carry
        mid = (lo & hi) + ((lo ^ hi) >> 1)
        cnt = jnp.sum((bits >= mid).astype(jnp.int32), axis=0, keepdims=True)
        ge = cnt >= _KSEL
        return jnp.where(ge, mid, lo), jnp.where(ge, hi, mid)

    lo, _ = jax.lax.fori_loop(0, 31, body, (lo0, hi0))
    gt = bits > lo
    eq = ((bits == lo) & causal).astype(jnp.int32)
    csum = eq
    for sh in range(11):
        rolled = jnp.roll(csum, 1 << sh, axis=0)
        csum = csum + jnp.where(kpos >= (1 << sh), rolled, 0)
    need = _KSEL - jnp.sum((gt & causal).astype(jnp.int32), axis=0,
                           keepdims=True)
    selmask = (gt | ((eq > 0) & (csum <= need))) & causal

    q = q_ref[...]
    k = k_ref[...]
    v = v_ref[...]
    go = go_ref[...]
    scale = jnp.float32(1.0 / math.sqrt(_DH))
    neg = jnp.float32(-jnp.inf)
    outs = []
    for h in range(_H):
        hkv = h // _NREP
        att = _mm_t(k[:, hkv * _DH:(hkv + 1) * _DH],
                    q[:, h * _DH:(h + 1) * _DH]) * scale
        att = jnp.where(selmask, att, neg)
        m = jnp.max(att, axis=0, keepdims=True)
        p = jnp.exp(att - m)
        s = jnp.sum(p, axis=0, keepdims=True)
        oh = jax.lax.dot_general(p, v[:, hkv * _DH:(hkv + 1) * _DH],
                                 (((0,), (0,)), ((), ())),
                                 preferred_element_type=jnp.float32)
        oh = oh * (go[:, h:h + 1] / jnp.transpose(s))
        outs.append(oh)
    o = jnp.concatenate(outs, axis=1)
    out_ref[...] = _mm(o, wo_ref[...])


def _full(shape):
    return pl.BlockSpec(shape, lambda i: (0,) * len(shape))


def _rows(width):
    return pl.BlockSpec((_BLK, width), lambda i: (i, 0))


def kernel(hidden_states, positions, Wq, Wk, Wv, Wo, Wqi, Wki, Wgi, Wgv,
           bgv, Wgo, bgo, interpret=False):
    del positions
    b, s, d = hidden_states.shape
    hs = hidden_states.reshape(s, d)
    grid = (s // _BLK,)

    q, k, v, qi, ki, gi, go = pl.pallas_call(
        _proj_kernel,
        grid=grid,
        in_specs=[
            _rows(_D), _full((_D, _H * _DH)), _full((_D, _HKV * _DH)),
            _full((_D, _HKV * _DH)), _full((_D, _HI * _DI)),
            _full((_D, _HI * _DI)), _full((_D, _HI)), _full((_D, _HKV)),
            _full((1, _HKV)), _full((_D, _H)), _full((1, _H)),
        ],
        out_specs=[
            _rows(_H * _DH), _rows(_HKV * _DH), _rows(_HKV * _DH),
            _rows(_HI * _DI), _rows(_HI * _DI), _rows(_HI), _rows(_H),
        ],
        out_shape=[
            jax.ShapeDtypeStruct((s, _H * _DH), jnp.float32),
            jax.ShapeDtypeStruct((s, _HKV * _DH), jnp.float32),
            jax.ShapeDtypeStruct((s, _HKV * _DH), jnp.float32),
            jax.ShapeDtypeStruct((s, _HI * _DI), jnp.float32),
            jax.ShapeDtypeStruct((s, _HI * _DI), jnp.float32),
            jax.ShapeDtypeStruct((s, _HI), jnp.float32),
            jax.ShapeDtypeStruct((s, _H), jnp.float32),
        ],
        compiler_params=pltpu.CompilerParams(
            dimension_semantics=("parallel",)),
        interpret=interpret,
    )(hs, Wq, Wk, Wv, Wqi, Wki, Wgi, Wgv, bgv.reshape(1, _HKV), Wgo,
      bgo.reshape(1, _H))

    out = pl.pallas_call(
        _attn_kernel,
        grid=grid,
        in_specs=[
            _rows(_H * _DH), _rows(_HI * _DI), _rows(_HI), _rows(_H),
            _full((s, _HI * _DI)), _full((s, _HKV * _DH)),
            _full((s, _HKV * _DH)), _full((_H * _DH, _D)),
        ],
        out_specs=_rows(_D),
        out_shape=jax.ShapeDtypeStruct((s, _D), jnp.float32),
        compiler_params=pltpu.CompilerParams(
            dimension_semantics=("parallel",)),
        interpret=interpret,
    )(q, qi, gi, go, ki, k, v, Wo)

    return out.reshape(b, s, d)
